# baseline (speedup 1.0000x reference)
import functools

import jax
import jax.numpy as jnp
from jax import lax
from jax.experimental import pallas as pl
from jax.experimental.pallas import tpu as pltpu
from jax.experimental.pallas import tpu_sc as plsc

NC = 2
NS = 16
NW = NC * NS
LANES = 16
BLK = 128


def _make_sc_kernel(batch, seq_len, dim):
    assert batch == NW * BLK and dim % LANES == 0 and seq_len % 2 == 0
    n_vregs = dim // LANES

    mesh = plsc.VectorSubcoreMesh(core_axis_name="c", subcore_axis_name="s")

    @functools.partial(
        pl.kernel,
        out_type=jax.ShapeDtypeStruct((seq_len * dim, batch), jnp.float32),
        mesh=mesh,
        scratch_types=[
            [pltpu.VMEM((1, BLK), jnp.int32) for _ in range(2)],
            [pltpu.VMEM((BLK,), jnp.int32) for _ in range(2)],
            [pltpu.VMEM((BLK, 2 * dim), jnp.float32) for _ in range(2)],
            [pltpu.VMEM((dim, BLK), jnp.float32) for _ in range(2)],
            pltpu.VMEM((seq_len, dim), jnp.float32),
            [pltpu.SemaphoreType.DMA for _ in range(2)],
            [pltpu.SemaphoreType.DMA for _ in range(2)],
        ],
        compiler_params=pltpu.CompilerParams(
            use_tc_tiling_on_sc=True, needs_layout_passes=False),
    )
    def sc_kernel(idx_hbm, tok_hbm, pos_hbm, out_hbm,
                  idx, idx2, rows, out_t, pos_v, gsem, ssem):
        wid = lax.axis_index("s") * NC + lax.axis_index("c")
        b0 = wid * BLK

        pltpu.sync_copy(pos_hbm, pos_v)

        def fetch_start(s, p):
            pltpu.sync_copy(idx_hbm.at[pl.ds(s, 1), pl.ds(b0, BLK)], idx[p])
            for g in range(BLK // LANES):
                sl = pl.ds(g * LANES, LANES)
                idx2[p][sl] = lax.shift_right_logical(idx[p][0, sl], 1)
            pltpu.async_copy(tok_hbm.at[idx2[p]], rows[p], gsem[p])

        def fetch_wait(p):
            pltpu.make_async_copy(tok_hbm.at[idx2[p]], rows[p], gsem[p]).wait()

        def store_start(s, p):
            pltpu.async_copy(
                out_t[p], out_hbm.at[pl.ds(s * dim, dim), pl.ds(b0, BLK)], ssem[p])

        def store_wait(s, p):
            pltpu.make_async_copy(
                out_t[p], out_hbm.at[pl.ds(s * dim, dim), pl.ds(b0, BLK)],
                ssem[p]).wait()

        def process(s, p):
            fetch_wait(p)
            s_splat = jnp.full((LANES,), s, jnp.int32)
            jvecs = [jnp.arange(LANES, dtype=jnp.int32) + g * LANES
                     for g in range(BLK // LANES)]
            hvecs = [(idx[p][0, pl.ds(g * LANES, LANES)] & 1) * dim
                     for g in range(BLK // LANES)]

            def d_body(d, carry):
                dsp = jnp.full((LANES,), d, jnp.int32)
                pd = plsc.load_gather(pos_v, [s_splat, dsp])
                for g in range(BLK // LANES):
                    val = plsc.load_gather(rows[p], [jvecs[g], hvecs[g] + d]) + pd
                    plsc.store_scatter(out_t[p], [dsp, jvecs[g]], val)
                return carry

            lax.fori_loop(0, dim, d_body, 0)
            store_start(s, p)

        fetch_start(0, 0)

        def loop_body(t, carry):
            for par in range(2):
                s = 2 * t + par
                pl.when(s + 1 < seq_len)(
                    functools.partial(fetch_start, s + 1, 1 - par))
                pl.when(s >= 2)(functools.partial(store_wait, s - 2, par))
                process(s, par)
            return carry

        lax.fori_loop(0, seq_len // 2, loop_body, 0)

        store_wait(seq_len - 2, 0)
        store_wait(seq_len - 1, 1)

    return sc_kernel


def kernel(inputs, token_table, position_table):
    batch, seq_len = inputs.shape
    vocab, dim = token_table.shape
    idx_t = inputs.T
    tok2 = token_table.reshape(vocab // 2, 2 * dim)
    sc = _make_sc_kernel(batch, seq_len, dim)
    out = sc(idx_t, tok2, position_table)
    return out.reshape(seq_len, dim, batch).transpose(2, 0, 1)


# speedup vs baseline: 1.6453x; 1.6453x over previous
import functools

import jax
import jax.numpy as jnp
from jax import lax
from jax.experimental import pallas as pl
from jax.experimental.pallas import tpu as pltpu
from jax.experimental.pallas import tpu_sc as plsc

NC = 2
NS = 16
NW = NC * NS
LANES = 16
BLK = 128


def _make_sc_kernel(batch, seq_len, dim):
    assert batch == NW * BLK and dim % LANES == 0 and seq_len % 2 == 0
    n_vregs = dim // LANES

    mesh = plsc.VectorSubcoreMesh(core_axis_name="c", subcore_axis_name="s")

    @functools.partial(
        pl.kernel,
        out_type=jax.ShapeDtypeStruct((seq_len * dim, batch), jnp.float32),
        mesh=mesh,
        scratch_types=[
            [pltpu.VMEM((1, BLK), jnp.int32) for _ in range(2)],
            [pltpu.VMEM((BLK,), jnp.int32) for _ in range(2)],
            [pltpu.VMEM((BLK, 2 * dim), jnp.float32) for _ in range(2)],
            [pltpu.VMEM((dim, BLK), jnp.float32) for _ in range(2)],
            pltpu.VMEM((seq_len, dim), jnp.float32),
            [pltpu.SemaphoreType.DMA for _ in range(2)],
            [pltpu.SemaphoreType.DMA for _ in range(2)],
        ],
        compiler_params=pltpu.CompilerParams(
            use_tc_tiling_on_sc=True, needs_layout_passes=False),
    )
    def sc_kernel(idx_hbm, tok_hbm, pos_hbm, out_hbm,
                  idx, idx2, rows, out_t, pos_v, gsem, ssem):
        wid = lax.axis_index("s") * NC + lax.axis_index("c")
        b0 = wid * BLK

        pltpu.sync_copy(pos_hbm, pos_v)

        def fetch_start(s, p):
            pltpu.sync_copy(idx_hbm.at[pl.ds(s, 1), pl.ds(b0, BLK)], idx[p])
            for g in range(BLK // LANES):
                sl = pl.ds(g * LANES, LANES)
                idx2[p][sl] = lax.shift_right_logical(idx[p][0, sl], 1)
            pltpu.async_copy(tok_hbm.at[idx2[p]], rows[p], gsem[p])

        def fetch_wait(p):
            pltpu.make_async_copy(tok_hbm.at[idx2[p]], rows[p], gsem[p]).wait()

        def store_start(s, p):
            pltpu.async_copy(
                out_t[p], out_hbm.at[pl.ds(s * dim, dim), pl.ds(b0, BLK)], ssem[p])

        def store_wait(s, p):
            pltpu.make_async_copy(
                out_t[p], out_hbm.at[pl.ds(s * dim, dim), pl.ds(b0, BLK)],
                ssem[p]).wait()

        def process(s, p):
            fetch_wait(p)
            s_splat = jnp.full((LANES,), s, jnp.int32)
            iota = jnp.arange(LANES, dtype=jnp.int32)
            jvecs = [iota + g * LANES for g in range(BLK // LANES)]
            hvecs = [(idx[p][0, pl.ds(g * LANES, LANES)] & 1) * dim
                     for g in range(BLK // LANES)]

            def k_body(k, carry):
                rot = (iota + k) & (LANES - 1)
                for c in range(n_vregs):
                    dvec = rot + c * LANES
                    pd = plsc.load_gather(pos_v, [s_splat, dvec])
                    for g in range(BLK // LANES):
                        val = plsc.load_gather(
                            rows[p], [jvecs[g], hvecs[g] + dvec]) + pd
                        plsc.store_scatter(out_t[p], [dvec, jvecs[g]], val)
                return carry

            lax.fori_loop(0, LANES, k_body, 0)
            store_start(s, p)

        fetch_start(0, 0)

        def loop_body(t, carry):
            for par in range(2):
                s = 2 * t + par
                pl.when(s + 1 < seq_len)(
                    functools.partial(fetch_start, s + 1, 1 - par))
                pl.when(s >= 2)(functools.partial(store_wait, s - 2, par))
                process(s, par)
            return carry

        lax.fori_loop(0, seq_len // 2, loop_body, 0)

        store_wait(seq_len - 2, 0)
        store_wait(seq_len - 1, 1)

    return sc_kernel


def kernel(inputs, token_table, position_table):
    batch, seq_len = inputs.shape
    vocab, dim = token_table.shape
    idx_t = inputs.T
    tok2 = token_table.reshape(vocab // 2, 2 * dim)
    sc = _make_sc_kernel(batch, seq_len, dim)
    out = sc(idx_t, tok2, position_table)
    return out.reshape(seq_len, dim, batch).transpose(2, 0, 1)


# speedup vs baseline: 1.9351x; 1.1761x over previous
import functools

import jax
import jax.numpy as jnp
from jax import lax
from jax.experimental import pallas as pl
from jax.experimental.pallas import tpu as pltpu
from jax.experimental.pallas import tpu_sc as plsc

NC = 2
NS = 16
NW = NC * NS
LANES = 16
BLK = 128


def _make_sc_kernel(batch, seq_len, dim):
    assert batch == NW * BLK and dim % LANES == 0
    n_vregs = dim // LANES

    mesh = plsc.VectorSubcoreMesh(core_axis_name="c", subcore_axis_name="s")

    @functools.partial(
        pl.kernel,
        out_type=jax.ShapeDtypeStruct((seq_len * dim, batch), jnp.float32),
        mesh=mesh,
        scratch_types=[
            pltpu.VMEM((seq_len, BLK), jnp.int32),
            [pltpu.VMEM((BLK, 2 * dim), jnp.float32) for _ in range(2)],
            [pltpu.VMEM((dim, BLK), jnp.float32) for _ in range(2)],
            pltpu.VMEM((seq_len, dim), jnp.float32),
            [pltpu.SemaphoreType.DMA for _ in range(2)],
            [pltpu.SemaphoreType.DMA for _ in range(2)],
        ],
        compiler_params=pltpu.CompilerParams(
            use_tc_tiling_on_sc=True, needs_layout_passes=False),
    )
    def sc_kernel(idx_hbm, tok_hbm, pos_hbm, out_hbm,
                  idx_all, rows, out_t, pos_v, gsem, ssem):
        wid = lax.axis_index("s") * NC + lax.axis_index("c")
        b0 = wid * BLK

        pltpu.sync_copy(pos_hbm, pos_v)
        pltpu.sync_copy(idx_hbm.at[:, pl.ds(b0, BLK)], idx_all)

        def fetch_start(s, p):
            pltpu.async_copy(tok_hbm.at[idx_all.at[s]], rows[p], gsem[p])

        def fetch_wait(p):
            pltpu.make_async_copy(tok_hbm.at[idx_all.at[0]], rows[p],
                                  gsem[p]).wait()

        def store_start(s, p):
            pltpu.async_copy(
                out_t[p], out_hbm.at[pl.ds(s * dim, dim), pl.ds(b0, BLK)], ssem[p])

        def store_wait(s, p):
            pltpu.make_async_copy(
                out_t[p], out_hbm.at[pl.ds(s * dim, dim), pl.ds(b0, BLK)],
                ssem[p]).wait()

        def process(s, p):
            fetch_wait(p)
            s_splat = jnp.full((LANES,), s, jnp.int32)
            iota = jnp.arange(LANES, dtype=jnp.int32)
            jvecs = [iota + g * LANES for g in range(BLK // LANES)]

            def k_body(k, carry):
                rot = (iota + k) & (LANES - 1)
                for c in range(n_vregs):
                    dvec = rot + c * LANES
                    pd = plsc.load_gather(pos_v, [s_splat, dvec])
                    for g in range(BLK // LANES):
                        val = plsc.load_gather(rows[p], [jvecs[g], dvec]) + pd
                        plsc.store_scatter(out_t[p], [dvec, jvecs[g]], val)
                return carry

            lax.fori_loop(0, LANES, k_body, 0)
            store_start(s, p)

        fetch_start(0, 0)

        def loop_body(t, carry):
            for par in range(2):
                s = 2 * t + par
                pl.when(s + 1 < seq_len)(
                    functools.partial(fetch_start, s + 1, 1 - par))
                pl.when(s >= 2)(functools.partial(store_wait, s - 2, par))
                process(s, par)
            return carry

        lax.fori_loop(0, seq_len // 2, loop_body, 0)

        store_wait(seq_len - 2, 0)
        store_wait(seq_len - 1, 1)

    return sc_kernel


def kernel(inputs, token_table, position_table):
    batch, seq_len = inputs.shape
    vocab, dim = token_table.shape
    idx_t = inputs.T
    tok_pad = jnp.pad(token_table, ((0, 0), (0, dim)))
    sc = _make_sc_kernel(batch, seq_len, dim)
    out = sc(idx_t, tok_pad, position_table)
    return out.reshape(seq_len, dim, batch).transpose(2, 0, 1)
